# baseline (speedup 1.0000x reference)
import functools

import jax
import jax.numpy as jnp
from jax import lax
from jax.experimental import pallas as pl
from jax.experimental.pallas import tpu as pltpu
from jax.experimental.pallas import tpu_sc as plsc

N = 10000
E = 320000
F_IN = 128
HEADS = 8
HID = 16
NCLS = 40

NC = 2
NS = 16
NW = NC * NS
LANES = 16

PADN = 16
NP_ = N + PADN
E_TOT = E + N
ROWS_PER_TILE = NP_ // NS

BN = 2000


def _sc_edge_pass(F, K):
    FW = F + 16
    G = F // 16
    D = 3
    E_PAD = ((E_TOT + D * NW * K - 1) // (D * NW * K)) * (D * NW * K)
    EPW = E_PAD // NW
    NCHUNK = EPW // K

    mesh = plsc.VectorSubcoreMesh(core_axis_name="c", subcore_axis_name="s")

    @functools.partial(
        pl.kernel,
        out_type=jax.ShapeDtypeStruct((NC, NP_, FW), jnp.float32),
        mesh=mesh,
        compiler_params=pltpu.CompilerParams(use_tc_tiling_on_sc=False),
        scratch_types=(
            [pltpu.VMEM((K,), jnp.int32) for _ in range(D)]
            + [pltpu.VMEM((K,), jnp.int32) for _ in range(D)]
            + [pltpu.VMEM((K,), jnp.int32) for _ in range(D)]
            + [pltpu.VMEM((K, FW), jnp.float32) for _ in range(D)]
            + [pltpu.VMEM((K, 16), jnp.float32) for _ in range(D)]
            + [pltpu.VMEM_SHARED((NP_, FW), jnp.float32)]
            + [pltpu.SemaphoreType.DMA for _ in range(3 * D)]
        ),
    )
    def edge_kernel(feat_hbm, src_hbm, dst_hbm, adst_hbm, out_hbm, *bufs):
        sidx = bufs[0:D]
        didx = bufs[D:2 * D]
        dsc = bufs[2 * D:3 * D]
        rows = bufs[3 * D:4 * D]
        adr = bufs[4 * D:5 * D]
        acc = bufs[5 * D]
        semi = bufs[5 * D + 1:5 * D + 1 + D]
        semg = bufs[5 * D + 1 + D:5 * D + 1 + 2 * D]
        sems = bufs[5 * D + 1 + 2 * D:5 * D + 1 + 3 * D]

        c = lax.axis_index("c")
        s = lax.axis_index("s")
        wid = c * NS + s
        ebase = wid * EPW
        zeros16 = jnp.zeros((LANES,), jnp.float32)
        lane = lax.iota(jnp.int32, LANES)
        t0 = s * ROWS_PER_TILE

        def start_idx(b, g):
            pltpu.async_copy(src_hbm.at[pl.ds(ebase + g * K, K)],
                             sidx[b], semi[b])
            pltpu.async_copy(dst_hbm.at[pl.ds(ebase + g * K, K)],
                             didx[b], semi[b])

        def wait_idx(b, g):
            pltpu.make_async_copy(src_hbm.at[pl.ds(ebase + g * K, K)],
                                  sidx[b], semi[b]).wait()
            pltpu.make_async_copy(dst_hbm.at[pl.ds(ebase + g * K, K)],
                                  didx[b], semi[b]).wait()

        def start_gather(b):
            pltpu.async_copy(feat_hbm.at[sidx[b]], rows[b], semg[b])
            pltpu.async_copy(adst_hbm.at[didx[b]], adr[b], semg[b])

        def wait_gather(b):
            pltpu.make_async_copy(feat_hbm.at[sidx[b]], rows[b], semg[b]).wait()
            pltpu.make_async_copy(adst_hbm.at[didx[b]], adr[b], semg[b]).wait()

        def wait_scat(b):
            pltpu.make_async_copy(rows[b], acc.at[dsc[b]], sems[b]).wait()

        def compute(b):
            rb, db = rows[b], adr[b]

            @plsc.parallel_loop(0, K, 1, unroll=4)
            def _sloop(k):
                e = rb[k, pl.ds(F, 16)] + db[k, :]
                e = jnp.maximum(e, 0.0) + 0.2 * jnp.minimum(e, 0.0)
                wrow = jnp.where(lane < 8, jnp.exp(e), 0.0)
                rb[k, pl.ds(F, 16)] = wrow
                for j in range(G):
                    v = rb[k, pl.ds(16 * j, 16)]
                    rb[k, pl.ds(16 * j, 16)] = v * wrow[j]

        def zrow(r, _):
            for j in range(FW // 16):
                rows[0][r, pl.ds(16 * j, 16)] = zeros16
            return 0
        lax.fori_loop(0, K, zrow, 0)
        copy_chunks = [(o, min(K, ROWS_PER_TILE - o))
                       for o in range(0, ROWS_PER_TILE, K)]
        for o, sz in copy_chunks:
            pltpu.sync_copy(rows[0].at[pl.ds(0, sz)],
                            acc.at[pl.ds(t0 + o, sz)])
        plsc.subcore_barrier()

        start_idx(0, 0)
        wait_idx(0, 0)
        start_idx(1, 1)
        start_idx(2, 2)
        start_gather(0)
        wait_idx(1, 1)
        start_gather(1)

        def triple(i, _):
            for b in range(D):
                g = D * i + b
                b2 = (b + 2) % D

                wait_gather(b)
                for q in range(K // 16):
                    dsc[b][pl.ds(16 * q, 16)] = didx[b][pl.ds(16 * q, 16)]

                @pl.when(g + D < NCHUNK)
                def _():
                    start_idx(b, g + D)

                compute(b)

                @pl.when(g >= 1)
                def _():
                    wait_scat(b2)

                @pl.when(g + 2 < NCHUNK)
                def _():
                    wait_idx(b2, g + 2)
                    start_gather(b2)

                pltpu.async_copy(rows[b], acc.at[dsc[b]], sems[b], add=True)
            return 0
        lax.fori_loop(0, NCHUNK // D, triple, 0)
        wait_scat(D - 1)

        plsc.subcore_barrier()
        for o, sz in copy_chunks:
            pltpu.sync_copy(acc.at[pl.ds(t0 + o, sz)],
                            out_hbm.at[c, pl.ds(t0 + o, sz)])

    return edge_kernel


def _prep1(x, W1, A1s, A1d):
    def body(x_ref, w_ref, as_ref, ad_ref, hs_ref, d_ref):
        h = jnp.dot(x_ref[...], w_ref[...], preferred_element_type=jnp.float32)
        s = jnp.dot(h, as_ref[...], preferred_element_type=jnp.float32)
        hs_ref[...] = jnp.concatenate([h, s], axis=1)
        d_ref[...] = jnp.dot(h, ad_ref[...], preferred_element_type=jnp.float32)

    grid = (N // BN,)
    return pl.pallas_call(
        body,
        grid=grid,
        in_specs=[
            pl.BlockSpec((BN, F_IN), lambda i: (i, 0)),
            pl.BlockSpec((F_IN, HEADS * HID), lambda i: (0, 0)),
            pl.BlockSpec((F_IN, 16), lambda i: (0, 0)),
            pl.BlockSpec((F_IN, 16), lambda i: (0, 0)),
        ],
        out_specs=[
            pl.BlockSpec((BN, HEADS * HID + 16), lambda i: (i, 0)),
            pl.BlockSpec((BN, 16), lambda i: (i, 0)),
        ],
        out_shape=[
            jax.ShapeDtypeStruct((N, HEADS * HID + 16), jnp.float32),
            jax.ShapeDtypeStruct((N, 16), jnp.float32),
        ],
    )(x, W1, A1s, A1d)


def _fin1(acc1, EXP8, b1, W2, a2s8, a2d8):
    def body(m0_ref, m1_ref, e8_ref, b1_ref, w2_ref, s8_ref, d8_ref,
             h2_ref, d2_ref):
        m = m0_ref[...] + m1_ref[...]
        num = m[:, 0:128]
        den8 = m[:, 128:136]
        den = jnp.dot(den8, e8_ref[...], preferred_element_type=jnp.float32)
        o = num / (den + 1e-16) + b1_ref[...]
        x2 = jnp.where(o > 0, o, jnp.exp(o) - 1.0)
        h2 = jnp.dot(x2, w2_ref[...], preferred_element_type=jnp.float32)
        s2 = jnp.dot(h2, s8_ref[...], preferred_element_type=jnp.float32)
        h2_ref[...] = jnp.concatenate(
            [h2, jnp.zeros((BN, 8), jnp.float32), s2], axis=1)
        d2_ref[...] = jnp.dot(h2, d8_ref[...], preferred_element_type=jnp.float32)

    grid = (N // BN,)
    FW1 = HEADS * HID + 16
    return pl.pallas_call(
        body,
        grid=grid,
        in_specs=[
            pl.BlockSpec((BN, FW1), lambda i: (i, 0)),
            pl.BlockSpec((BN, FW1), lambda i: (i, 0)),
            pl.BlockSpec((8, 128), lambda i: (0, 0)),
            pl.BlockSpec((1, 128), lambda i: (0, 0)),
            pl.BlockSpec((128, NCLS), lambda i: (0, 0)),
            pl.BlockSpec((NCLS, 16), lambda i: (0, 0)),
            pl.BlockSpec((NCLS, 16), lambda i: (0, 0)),
        ],
        out_specs=[
            pl.BlockSpec((BN, 64), lambda i: (i, 0)),
            pl.BlockSpec((BN, 16), lambda i: (i, 0)),
        ],
        out_shape=[
            jax.ShapeDtypeStruct((N, 64), jnp.float32),
            jax.ShapeDtypeStruct((N, 16), jnp.float32),
        ],
    )(acc1[0], acc1[1], EXP8, b1, W2, a2s8, a2d8)


def _fin2(acc2, b2):
    def body(m0_ref, m1_ref, b2_ref, o_ref):
        m = m0_ref[...] + m1_ref[...]
        num = m[:, 0:NCLS]
        den = m[:, 48:49]
        o_ref[...] = num / (den + 1e-16) + b2_ref[...]

    grid = (N // BN,)
    return pl.pallas_call(
        body,
        grid=grid,
        in_specs=[
            pl.BlockSpec((BN, 64), lambda i: (i, 0)),
            pl.BlockSpec((BN, 64), lambda i: (i, 0)),
            pl.BlockSpec((1, NCLS), lambda i: (0, 0)),
        ],
        out_specs=pl.BlockSpec((BN, NCLS), lambda i: (i, 0)),
        out_shape=jax.ShapeDtypeStruct((N, NCLS), jnp.float32),
    )(acc2[0], acc2[1], b2)


def kernel(x, edge_index, W1, att_src1, att_dst1, b1, W2, att_src2,
           att_dst2, b2):
    loop = jnp.arange(N, dtype=jnp.int32)
    gran = 3 * NW * 192
    e_pad_max = ((E_TOT + gran - 1) // gran) * gran
    npad_e = e_pad_max - E_TOT
    pad_idx = N + (jnp.arange(npad_e, dtype=jnp.int32) % PADN)
    src = jnp.concatenate([edge_index[0], loop, pad_idx])
    dst = jnp.concatenate([edge_index[1], loop, pad_idx])

    eye8 = jnp.eye(8, dtype=jnp.float32)
    z1288 = jnp.zeros((128, 8), jnp.float32)
    A1s = jnp.concatenate(
        [(att_src1[:, :, None] * eye8[:, None, :]).reshape(128, 8), z1288],
        axis=1)
    A1d = jnp.concatenate(
        [(att_dst1[:, :, None] * eye8[:, None, :]).reshape(128, 8), z1288],
        axis=1)
    EXP8 = jnp.repeat(eye8, 16, axis=1)
    z408 = jnp.zeros((NCLS, 8), jnp.float32)
    a2s8 = jnp.concatenate(
        [jnp.tile(att_src2.reshape(NCLS, 1), (1, 8)), z408], axis=1)
    a2d8 = jnp.concatenate(
        [jnp.tile(att_dst2.reshape(NCLS, 1), (1, 8)), z408], axis=1)

    hs1, adst1 = _prep1(x, W1, A1s, A1d)
    zpadFW = jnp.zeros((PADN, HEADS * HID + 16), jnp.float32)
    zpad16 = jnp.zeros((PADN, 16), jnp.float32)
    hs1p = jnp.concatenate([hs1, zpadFW], axis=0)
    adst1p = jnp.concatenate([adst1, zpad16], axis=0)

    acc1 = _sc_edge_pass(HEADS * HID, 80)(hs1p, src, dst, adst1p)

    hs2, adst2 = _fin1(acc1[:, :N], EXP8, b1.reshape(1, 128),
                       W2, a2s8, a2d8)
    hs2p = jnp.concatenate([hs2, jnp.zeros((PADN, 64), jnp.float32)], axis=0)
    adst2p = jnp.concatenate([adst2, zpad16], axis=0)

    acc2 = _sc_edge_pass(48, 192)(hs2p, src, dst, adst2p)

    return _fin2(acc2[:, :N], b2.reshape(1, NCLS))


# speedup vs baseline: 1.0506x; 1.0506x over previous
import functools

import jax
import jax.numpy as jnp
from jax import lax
from jax.experimental import pallas as pl
from jax.experimental.pallas import tpu as pltpu
from jax.experimental.pallas import tpu_sc as plsc

N = 10000
E = 320000
F_IN = 128
HEADS = 8
HID = 16
NCLS = 40

NC = 2
NS = 16
NW = NC * NS
LANES = 16

PADN = 16
NP_ = N + PADN
E_TOT = E + N
ROWS_PER_TILE = NP_ // NS

BN = 2000


def _sc_edge_pass(F, K):
    FW = F + 16
    G = F // 16
    D = 3
    EPW = E // NW
    NCHUNK = EPW // K
    assert EPW % K == 0 and K % 16 == 0
    NFULL = (NCHUNK // D) * D
    REM = NCHUNK - NFULL

    mesh = plsc.VectorSubcoreMesh(core_axis_name="c", subcore_axis_name="s")

    @functools.partial(
        pl.kernel,
        out_type=jax.ShapeDtypeStruct((NC, NP_, FW), jnp.float32),
        mesh=mesh,
        compiler_params=pltpu.CompilerParams(use_tc_tiling_on_sc=False),
        scratch_types=(
            [pltpu.VMEM((K,), jnp.int32) for _ in range(D)]
            + [pltpu.VMEM((K,), jnp.int32) for _ in range(D)]
            + [pltpu.VMEM((K,), jnp.int32) for _ in range(D)]
            + [pltpu.VMEM((K, FW), jnp.float32) for _ in range(D)]
            + [pltpu.VMEM((K, 16), jnp.float32) for _ in range(D)]
            + [pltpu.VMEM_SHARED((NP_, FW), jnp.float32)]
            + [pltpu.SemaphoreType.DMA for _ in range(3 * D)]
        ),
    )
    def edge_kernel(feat_hbm, ei_hbm, adst_hbm, out_hbm, *bufs):
        sidx = bufs[0:D]
        didx = bufs[D:2 * D]
        dsc = bufs[2 * D:3 * D]
        rows = bufs[3 * D:4 * D]
        adr = bufs[4 * D:5 * D]
        acc = bufs[5 * D]
        semi = bufs[5 * D + 1:5 * D + 1 + D]
        semg = bufs[5 * D + 1 + D:5 * D + 1 + 2 * D]
        sems = bufs[5 * D + 1 + 2 * D:5 * D + 1 + 3 * D]

        c = lax.axis_index("c")
        s = lax.axis_index("s")
        wid = c * NS + s
        ebase = wid * EPW
        zeros16 = jnp.zeros((LANES,), jnp.float32)
        lane = lax.iota(jnp.int32, LANES)
        t0 = s * ROWS_PER_TILE

        def start_idx(b, g):
            pltpu.async_copy(ei_hbm.at[0, pl.ds(ebase + g * K, K)],
                             sidx[b], semi[b])
            pltpu.async_copy(ei_hbm.at[1, pl.ds(ebase + g * K, K)],
                             didx[b], semi[b])

        def wait_idx(b, g):
            pltpu.make_async_copy(ei_hbm.at[0, pl.ds(ebase + g * K, K)],
                                  sidx[b], semi[b]).wait()
            pltpu.make_async_copy(ei_hbm.at[1, pl.ds(ebase + g * K, K)],
                                  didx[b], semi[b]).wait()

        def start_gather(b):
            pltpu.async_copy(feat_hbm.at[sidx[b]], rows[b], semg[b])
            pltpu.async_copy(adst_hbm.at[didx[b]], adr[b], semg[b])

        def wait_gather(b):
            pltpu.make_async_copy(feat_hbm.at[sidx[b]], rows[b], semg[b]).wait()
            pltpu.make_async_copy(adst_hbm.at[didx[b]], adr[b], semg[b]).wait()

        def wait_scat(b):
            pltpu.make_async_copy(rows[b], acc.at[dsc[b]], sems[b]).wait()

        def compute(b):
            rb, db = rows[b], adr[b]

            @plsc.parallel_loop(0, K, 1, unroll=4)
            def _sloop(k):
                e = rb[k, pl.ds(F, 16)] + db[k, :]
                e = jnp.maximum(e, 0.0) + 0.2 * jnp.minimum(e, 0.0)
                wrow = jnp.where(lane < 8, jnp.exp(e), 0.0)
                rb[k, pl.ds(F, 16)] = wrow
                for j in range(G):
                    v = rb[k, pl.ds(16 * j, 16)]
                    rb[k, pl.ds(16 * j, 16)] = v * wrow[j]

        def do_chunk(b, g, static_g=None):
            gg = g if static_g is None else static_g
            b2 = (b + 2) % D

            wait_gather(b)
            for q in range(K // 16):
                dsc[b][pl.ds(16 * q, 16)] = didx[b][pl.ds(16 * q, 16)]

            def _prefetch_idx():
                start_idx(b, gg + D)
            if static_g is None:
                pl.when(g + D < NCHUNK)(_prefetch_idx)
            elif static_g + D < NCHUNK:
                _prefetch_idx()

            compute(b)

            def _drain_scat():
                wait_scat(b2)
            if static_g is None:
                pl.when(g >= 1)(_drain_scat)
            elif static_g >= 1:
                _drain_scat()

            def _next_gather():
                wait_idx(b2, gg + 2)
                start_gather(b2)
            if static_g is None:
                pl.when(g + 2 < NCHUNK)(_next_gather)
            elif static_g + 2 < NCHUNK:
                _next_gather()

            pltpu.async_copy(rows[b], acc.at[dsc[b]], sems[b], add=True)

        def zrow(r, _):
            for j in range(FW // 16):
                rows[0][r, pl.ds(16 * j, 16)] = zeros16
            return 0
        lax.fori_loop(0, K, zrow, 0)
        copy_chunks = [(o, min(K, ROWS_PER_TILE - o))
                       for o in range(0, ROWS_PER_TILE, K)]
        for o, sz in copy_chunks:
            pltpu.sync_copy(rows[0].at[pl.ds(0, sz)],
                            acc.at[pl.ds(t0 + o, sz)])
        plsc.subcore_barrier()

        start_idx(0, 0)
        wait_idx(0, 0)
        start_idx(1, 1)
        start_idx(2, 2)
        start_gather(0)
        wait_idx(1, 1)
        start_gather(1)

        def triple(i, _):
            for b in range(D):
                do_chunk(b, D * i + b)
            return 0
        lax.fori_loop(0, NFULL // D, triple, 0)
        for r in range(REM):
            do_chunk((NFULL + r) % D, NFULL + r, static_g=NFULL + r)
        wait_scat((NCHUNK - 1) % D)

        plsc.subcore_barrier()
        for o, sz in copy_chunks:
            pltpu.sync_copy(acc.at[pl.ds(t0 + o, sz)],
                            out_hbm.at[c, pl.ds(t0 + o, sz)])

    return edge_kernel


def _prep1(x, W1, A1s, A1d):
    def body(x_ref, w_ref, as_ref, ad_ref, hs_ref, d_ref):
        h = jnp.dot(x_ref[...], w_ref[...], preferred_element_type=jnp.float32)
        s = jnp.dot(h, as_ref[...], preferred_element_type=jnp.float32)
        hs_ref[...] = jnp.concatenate([h, s], axis=1)
        d_ref[...] = jnp.dot(h, ad_ref[...], preferred_element_type=jnp.float32)

    grid = (N // BN,)
    return pl.pallas_call(
        body,
        grid=grid,
        in_specs=[
            pl.BlockSpec((BN, F_IN), lambda i: (i, 0)),
            pl.BlockSpec((F_IN, HEADS * HID), lambda i: (0, 0)),
            pl.BlockSpec((F_IN, 16), lambda i: (0, 0)),
            pl.BlockSpec((F_IN, 16), lambda i: (0, 0)),
        ],
        out_specs=[
            pl.BlockSpec((BN, HEADS * HID + 16), lambda i: (i, 0)),
            pl.BlockSpec((BN, 16), lambda i: (i, 0)),
        ],
        out_shape=[
            jax.ShapeDtypeStruct((N, HEADS * HID + 16), jnp.float32),
            jax.ShapeDtypeStruct((N, 16), jnp.float32),
        ],
    )(x, W1, A1s, A1d)


def _fin1(acc1, hs1, adst1, EXP8, b1, W2, a2s8, a2d8):
    def body(m0_ref, m1_ref, hs_ref, ad_ref, e8_ref,
             b1_ref, w2_ref, s8_ref, d8_ref, h2_ref, d2_ref):
        m = m0_ref[...] + m1_ref[...]
        h1 = hs_ref[:, 0:128]
        asrc = hs_ref[:, 128:136]
        adst = ad_ref[:, 0:8]
        es = asrc + adst
        ws = jnp.exp(jnp.maximum(es, 0.0) + 0.2 * jnp.minimum(es, 0.0))
        ws128 = jnp.dot(ws, e8_ref[...], preferred_element_type=jnp.float32)
        num = m[:, 0:128] + ws128 * h1
        den8 = m[:, 128:136] + ws
        den = jnp.dot(den8, e8_ref[...], preferred_element_type=jnp.float32)
        o = num / (den + 1e-16) + b1_ref[...]
        x2 = jnp.where(o > 0, o, jnp.exp(o) - 1.0)
        h2 = jnp.dot(x2, w2_ref[...], preferred_element_type=jnp.float32)
        s2 = jnp.dot(h2, s8_ref[...], preferred_element_type=jnp.float32)
        h2_ref[...] = jnp.concatenate(
            [h2, jnp.zeros((BN, 8), jnp.float32), s2], axis=1)
        d2_ref[...] = jnp.dot(h2, d8_ref[...], preferred_element_type=jnp.float32)

    grid = (N // BN,)
    FW1 = HEADS * HID + 16
    return pl.pallas_call(
        body,
        grid=grid,
        in_specs=[
            pl.BlockSpec((BN, FW1), lambda i: (i, 0)),
            pl.BlockSpec((BN, FW1), lambda i: (i, 0)),
            pl.BlockSpec((BN, FW1), lambda i: (i, 0)),
            pl.BlockSpec((BN, 16), lambda i: (i, 0)),
            pl.BlockSpec((8, 128), lambda i: (0, 0)),
            pl.BlockSpec((1, 128), lambda i: (0, 0)),
            pl.BlockSpec((128, NCLS), lambda i: (0, 0)),
            pl.BlockSpec((NCLS, 16), lambda i: (0, 0)),
            pl.BlockSpec((NCLS, 16), lambda i: (0, 0)),
        ],
        out_specs=[
            pl.BlockSpec((BN, 64), lambda i: (i, 0)),
            pl.BlockSpec((BN, 16), lambda i: (i, 0)),
        ],
        out_shape=[
            jax.ShapeDtypeStruct((N, 64), jnp.float32),
            jax.ShapeDtypeStruct((N, 16), jnp.float32),
        ],
    )(acc1[0], acc1[1], hs1, adst1, EXP8, b1, W2, a2s8, a2d8)


def _fin2(acc2, hs2, adst2, b2):
    def body(m0_ref, m1_ref, hs_ref, ad_ref, b2_ref, o_ref):
        m = m0_ref[...] + m1_ref[...]
        h2 = hs_ref[:, 0:NCLS]
        es = hs_ref[:, 48:49] + ad_ref[:, 0:1]
        ws = jnp.exp(jnp.maximum(es, 0.0) + 0.2 * jnp.minimum(es, 0.0))
        num = m[:, 0:NCLS] + ws * h2
        den = m[:, 48:49] + ws
        o_ref[...] = num / (den + 1e-16) + b2_ref[...]

    grid = (N // BN,)
    return pl.pallas_call(
        body,
        grid=grid,
        in_specs=[
            pl.BlockSpec((BN, 64), lambda i: (i, 0)),
            pl.BlockSpec((BN, 64), lambda i: (i, 0)),
            pl.BlockSpec((BN, 64), lambda i: (i, 0)),
            pl.BlockSpec((BN, 16), lambda i: (i, 0)),
            pl.BlockSpec((1, NCLS), lambda i: (0, 0)),
        ],
        out_specs=pl.BlockSpec((BN, NCLS), lambda i: (i, 0)),
        out_shape=jax.ShapeDtypeStruct((N, NCLS), jnp.float32),
    )(acc2[0], acc2[1], hs2, adst2, b2)


def kernel(x, edge_index, W1, att_src1, att_dst1, b1, W2, att_src2,
           att_dst2, b2):
    eye8 = jnp.eye(8, dtype=jnp.float32)
    z1288 = jnp.zeros((128, 8), jnp.float32)
    A1s = jnp.concatenate(
        [(att_src1[:, :, None] * eye8[:, None, :]).reshape(128, 8), z1288],
        axis=1)
    A1d = jnp.concatenate(
        [(att_dst1[:, :, None] * eye8[:, None, :]).reshape(128, 8), z1288],
        axis=1)
    EXP8 = jnp.repeat(eye8, 16, axis=1)
    z408 = jnp.zeros((NCLS, 8), jnp.float32)
    a2s8 = jnp.concatenate(
        [jnp.tile(att_src2.reshape(NCLS, 1), (1, 8)), z408], axis=1)
    a2d8 = jnp.concatenate(
        [jnp.tile(att_dst2.reshape(NCLS, 1), (1, 8)), z408], axis=1)

    hs1, adst1 = _prep1(x, W1, A1s, A1d)
    acc1 = _sc_edge_pass(HEADS * HID, 80)(hs1, edge_index, adst1)

    hs2, adst2 = _fin1(acc1, hs1, adst1, EXP8, b1.reshape(1, 128),
                       W2, a2s8, a2d8)
    acc2 = _sc_edge_pass(48, 80)(hs2, edge_index, adst2)

    return _fin2(acc2, hs2, adst2, b2.reshape(1, NCLS))


# speedup vs baseline: 1.1701x; 1.1138x over previous
import functools

import jax
import jax.numpy as jnp
from jax import lax
from jax.experimental import pallas as pl
from jax.experimental.pallas import tpu as pltpu
from jax.experimental.pallas import tpu_sc as plsc

N = 10000
E = 320000
F_IN = 128
HEADS = 8
HID = 16
NCLS = 40

NC = 2
NS = 16
NW = NC * NS
LANES = 16

PADN = 16
NP_ = N + PADN
E_TOT = E + N
ROWS_PER_TILE = NP_ // NS

BN = 2000


def _sc_edge_pass(F, K):
    FW = F + 16
    G = F // 16
    D = 3
    EPW = E // NW
    NCHUNK = EPW // K
    assert EPW % K == 0 and K % 8 == 0 and K >= 16
    DSC_OFFS = list(range(0, K - 15, 16))
    if K % 16:
        DSC_OFFS.append(K - 16)
    NFULL = (NCHUNK // D) * D
    REM = NCHUNK - NFULL

    mesh = plsc.VectorSubcoreMesh(core_axis_name="c", subcore_axis_name="s")

    @functools.partial(
        pl.kernel,
        out_type=jax.ShapeDtypeStruct((NC, NP_, FW), jnp.float32),
        mesh=mesh,
        compiler_params=pltpu.CompilerParams(use_tc_tiling_on_sc=False),
        scratch_types=(
            [pltpu.VMEM((K,), jnp.int32) for _ in range(D)]
            + [pltpu.VMEM((K,), jnp.int32) for _ in range(D)]
            + [pltpu.VMEM((K,), jnp.int32) for _ in range(D)]
            + [pltpu.VMEM((K, FW), jnp.float32) for _ in range(D)]
            + [pltpu.VMEM((K, 16), jnp.float32) for _ in range(D)]
            + [pltpu.VMEM_SHARED((NP_, FW), jnp.float32)]
            + [pltpu.SemaphoreType.DMA for _ in range(3 * D)]
        ),
    )
    def edge_kernel(feat_hbm, ei_hbm, adst_hbm, out_hbm, *bufs):
        sidx = bufs[0:D]
        didx = bufs[D:2 * D]
        dsc = bufs[2 * D:3 * D]
        rows = bufs[3 * D:4 * D]
        adr = bufs[4 * D:5 * D]
        acc = bufs[5 * D]
        semi = bufs[5 * D + 1:5 * D + 1 + D]
        semg = bufs[5 * D + 1 + D:5 * D + 1 + 2 * D]
        sems = bufs[5 * D + 1 + 2 * D:5 * D + 1 + 3 * D]

        c = lax.axis_index("c")
        s = lax.axis_index("s")
        wid = c * NS + s
        ebase = wid * EPW
        zeros16 = jnp.zeros((LANES,), jnp.float32)
        lane = lax.iota(jnp.int32, LANES)
        t0 = s * ROWS_PER_TILE

        def start_idx(b, g):
            pltpu.async_copy(ei_hbm.at[0, pl.ds(ebase + g * K, K)],
                             sidx[b], semi[b])
            pltpu.async_copy(ei_hbm.at[1, pl.ds(ebase + g * K, K)],
                             didx[b], semi[b])

        def wait_idx(b, g):
            pltpu.make_async_copy(ei_hbm.at[0, pl.ds(ebase + g * K, K)],
                                  sidx[b], semi[b]).wait()
            pltpu.make_async_copy(ei_hbm.at[1, pl.ds(ebase + g * K, K)],
                                  didx[b], semi[b]).wait()

        def start_gather(b):
            pltpu.async_copy(feat_hbm.at[sidx[b]], rows[b], semg[b])
            pltpu.async_copy(adst_hbm.at[didx[b]], adr[b], semg[b])

        def wait_gather(b):
            pltpu.make_async_copy(feat_hbm.at[sidx[b]], rows[b], semg[b]).wait()
            pltpu.make_async_copy(adst_hbm.at[didx[b]], adr[b], semg[b]).wait()

        def wait_scat(b):
            pltpu.make_async_copy(rows[b], acc.at[dsc[b]], sems[b]).wait()

        def compute(b):
            rb, db = rows[b], adr[b]

            @plsc.parallel_loop(0, K, 1, unroll=4)
            def _sloop(k):
                e = rb[k, pl.ds(F, 16)] + db[k, :]
                e = jnp.maximum(e, 0.0) + 0.2 * jnp.minimum(e, 0.0)
                wrow = jnp.where(lane < 8, jnp.exp(e), 0.0)
                rb[k, pl.ds(F, 16)] = wrow
                for j in range(G):
                    v = rb[k, pl.ds(16 * j, 16)]
                    rb[k, pl.ds(16 * j, 16)] = v * wrow[j]

        def do_chunk(b, g, static_g=None):
            gg = g if static_g is None else static_g
            b2 = (b + 2) % D

            wait_gather(b)
            for o in DSC_OFFS:
                dsc[b][pl.ds(o, 16)] = didx[b][pl.ds(o, 16)]

            def _prefetch_idx():
                start_idx(b, gg + D)
            if static_g is None:
                pl.when(g + D < NCHUNK)(_prefetch_idx)
            elif static_g + D < NCHUNK:
                _prefetch_idx()

            compute(b)

            def _drain_scat():
                wait_scat(b2)
            if static_g is None:
                pl.when(g >= 1)(_drain_scat)
            elif static_g >= 1:
                _drain_scat()

            def _next_gather():
                wait_idx(b2, gg + 2)
                start_gather(b2)
            if static_g is None:
                pl.when(g + 2 < NCHUNK)(_next_gather)
            elif static_g + 2 < NCHUNK:
                _next_gather()

            pltpu.async_copy(rows[b], acc.at[dsc[b]], sems[b], add=True)

        def zrow(r, _):
            for j in range(FW // 16):
                rows[0][r, pl.ds(16 * j, 16)] = zeros16
            return 0
        lax.fori_loop(0, K, zrow, 0)
        copy_chunks = [(o, min(K, ROWS_PER_TILE - o))
                       for o in range(0, ROWS_PER_TILE, K)]
        for o, sz in copy_chunks:
            pltpu.sync_copy(rows[0].at[pl.ds(0, sz)],
                            acc.at[pl.ds(t0 + o, sz)])
        plsc.subcore_barrier()

        start_idx(0, 0)
        wait_idx(0, 0)
        start_idx(1, 1)
        start_idx(2, 2)
        start_gather(0)
        wait_idx(1, 1)
        start_gather(1)

        def triple(i, _):
            for b in range(D):
                do_chunk(b, D * i + b)
            return 0
        lax.fori_loop(0, NFULL // D, triple, 0)
        for r in range(REM):
            do_chunk((NFULL + r) % D, NFULL + r, static_g=NFULL + r)
        wait_scat((NCHUNK - 1) % D)

        plsc.subcore_barrier()
        for o, sz in copy_chunks:
            pltpu.sync_copy(acc.at[pl.ds(t0 + o, sz)],
                            out_hbm.at[c, pl.ds(t0 + o, sz)])

    return edge_kernel


def _prep1(x, W1, A1s, A1d):
    def body(x_ref, w_ref, as_ref, ad_ref, hs_ref, d_ref):
        h = jnp.dot(x_ref[...], w_ref[...], preferred_element_type=jnp.float32)
        s = jnp.dot(h, as_ref[...], preferred_element_type=jnp.float32)
        hs_ref[...] = jnp.concatenate([h, s], axis=1)
        d_ref[...] = jnp.dot(h, ad_ref[...], preferred_element_type=jnp.float32)

    grid = (N // BN,)
    return pl.pallas_call(
        body,
        grid=grid,
        in_specs=[
            pl.BlockSpec((BN, F_IN), lambda i: (i, 0)),
            pl.BlockSpec((F_IN, HEADS * HID), lambda i: (0, 0)),
            pl.BlockSpec((F_IN, 16), lambda i: (0, 0)),
            pl.BlockSpec((F_IN, 16), lambda i: (0, 0)),
        ],
        out_specs=[
            pl.BlockSpec((BN, HEADS * HID + 16), lambda i: (i, 0)),
            pl.BlockSpec((BN, 16), lambda i: (i, 0)),
        ],
        out_shape=[
            jax.ShapeDtypeStruct((N, HEADS * HID + 16), jnp.float32),
            jax.ShapeDtypeStruct((N, 16), jnp.float32),
        ],
    )(x, W1, A1s, A1d)


def _fin1(acc1, hs1, adst1, EXP8, b1, W2, a2s8, a2d8):
    def body(m_ref, hs_ref, ad_ref, e8_ref,
             b1_ref, w2_ref, s8_ref, d8_ref, h2_ref, d2_ref):
        m = m_ref[0] + m_ref[1]
        h1 = hs_ref[:, 0:128]
        asrc = hs_ref[:, 128:136]
        adst = ad_ref[:, 0:8]
        es = asrc + adst
        ws = jnp.exp(jnp.maximum(es, 0.0) + 0.2 * jnp.minimum(es, 0.0))
        ws128 = jnp.dot(ws, e8_ref[...], preferred_element_type=jnp.float32)
        num = m[:, 0:128] + ws128 * h1
        den8 = m[:, 128:136] + ws
        den = jnp.dot(den8, e8_ref[...], preferred_element_type=jnp.float32)
        o = num / (den + 1e-16) + b1_ref[...]
        x2 = jnp.where(o > 0, o, jnp.exp(o) - 1.0)
        h2 = jnp.dot(x2, w2_ref[...], preferred_element_type=jnp.float32)
        s2 = jnp.dot(h2, s8_ref[...], preferred_element_type=jnp.float32)
        h2_ref[...] = jnp.concatenate(
            [h2, jnp.zeros((BN, 8), jnp.float32), s2], axis=1)
        d2_ref[...] = jnp.dot(h2, d8_ref[...], preferred_element_type=jnp.float32)

    grid = (N // BN,)
    FW1 = HEADS * HID + 16
    return pl.pallas_call(
        body,
        grid=grid,
        in_specs=[
            pl.BlockSpec((NC, BN, FW1), lambda i: (0, i, 0)),
            pl.BlockSpec((BN, FW1), lambda i: (i, 0)),
            pl.BlockSpec((BN, 16), lambda i: (i, 0)),
            pl.BlockSpec((8, 128), lambda i: (0, 0)),
            pl.BlockSpec((1, 128), lambda i: (0, 0)),
            pl.BlockSpec((128, NCLS), lambda i: (0, 0)),
            pl.BlockSpec((NCLS, 16), lambda i: (0, 0)),
            pl.BlockSpec((NCLS, 16), lambda i: (0, 0)),
        ],
        out_specs=[
            pl.BlockSpec((BN, 64), lambda i: (i, 0)),
            pl.BlockSpec((BN, 16), lambda i: (i, 0)),
        ],
        out_shape=[
            jax.ShapeDtypeStruct((N, 64), jnp.float32),
            jax.ShapeDtypeStruct((N, 16), jnp.float32),
        ],
    )(acc1, hs1, adst1, EXP8, b1, W2, a2s8, a2d8)


def _fin2(acc2, hs2, adst2, b2):
    def body(m_ref, hs_ref, ad_ref, b2_ref, o_ref):
        m = m_ref[0] + m_ref[1]
        h2 = hs_ref[:, 0:NCLS]
        es = hs_ref[:, 48:49] + ad_ref[:, 0:1]
        ws = jnp.exp(jnp.maximum(es, 0.0) + 0.2 * jnp.minimum(es, 0.0))
        num = m[:, 0:NCLS] + ws * h2
        den = m[:, 48:49] + ws
        o_ref[...] = num / (den + 1e-16) + b2_ref[...]

    grid = (N // BN,)
    return pl.pallas_call(
        body,
        grid=grid,
        in_specs=[
            pl.BlockSpec((NC, BN, 64), lambda i: (0, i, 0)),
            pl.BlockSpec((BN, 64), lambda i: (i, 0)),
            pl.BlockSpec((BN, 16), lambda i: (i, 0)),
            pl.BlockSpec((1, NCLS), lambda i: (0, 0)),
        ],
        out_specs=pl.BlockSpec((BN, NCLS), lambda i: (i, 0)),
        out_shape=jax.ShapeDtypeStruct((N, NCLS), jnp.float32),
    )(acc2, hs2, adst2, b2)


def kernel(x, edge_index, W1, att_src1, att_dst1, b1, W2, att_src2,
           att_dst2, b2):
    eye8 = jnp.eye(8, dtype=jnp.float32)
    z1288 = jnp.zeros((128, 8), jnp.float32)
    A1s = jnp.concatenate(
        [(att_src1[:, :, None] * eye8[:, None, :]).reshape(128, 8), z1288],
        axis=1)
    A1d = jnp.concatenate(
        [(att_dst1[:, :, None] * eye8[:, None, :]).reshape(128, 8), z1288],
        axis=1)
    EXP8 = jnp.repeat(eye8, 16, axis=1)
    z408 = jnp.zeros((NCLS, 8), jnp.float32)
    a2s8 = jnp.concatenate(
        [jnp.tile(att_src2.reshape(NCLS, 1), (1, 8)), z408], axis=1)
    a2d8 = jnp.concatenate(
        [jnp.tile(att_dst2.reshape(NCLS, 1), (1, 8)), z408], axis=1)

    hs1, adst1 = _prep1(x, W1, A1s, A1d)
    acc1 = _sc_edge_pass(HEADS * HID, 80)(hs1, edge_index, adst1)

    hs2, adst2 = _fin1(acc1, hs1, adst1, EXP8, b1.reshape(1, 128),
                       W2, a2s8, a2d8)
    acc2 = _sc_edge_pass(48, 200)(hs2, edge_index, adst2)

    return _fin2(acc2, hs2, adst2, b2.reshape(1, NCLS))
